# baseline (speedup 1.0000x reference)
import functools

import jax
import jax.numpy as jnp
from jax import lax
from jax.experimental import pallas as pl
from jax.experimental.pallas import tpu as pltpu
from jax.experimental.pallas import tpu_sc as plsc

N = 10000
E = 320000
EP = 327680
H = 128
HP = H // 2
NORM = 100.0

NC = 2
NS = 16
NW = NC * NS
EPW = EP // NW
CH = 80
NCH_G = EPW // CH

BE = 4096
RB = BE // 512
GROWS = EP // 512

RS = GROWS // NS
NCHS = RS * 4
RPT = 640


def _pre_node(h, W1a, W1b):
    def body(h_ref, wa_ref, wb_ref, a_ref, b_ref):
        hv = h_ref[...]
        a_ref[...] = jnp.dot(hv, wa_ref[...], preferred_element_type=jnp.float32)
        b_ref[...] = jnp.dot(hv, wb_ref[...], preferred_element_type=jnp.float32)

    BN = 2000
    return pl.pallas_call(
        body,
        grid=(N // BN,),
        in_specs=[
            pl.BlockSpec((BN, H), lambda i: (i, 0)),
            pl.BlockSpec((H, H), lambda i: (0, 0)),
            pl.BlockSpec((H, H), lambda i: (0, 0)),
        ],
        out_specs=[
            pl.BlockSpec((BN, H), lambda i: (i, 0)),
            pl.BlockSpec((BN, H), lambda i: (i, 0)),
        ],
        out_shape=[
            jax.ShapeDtypeStruct((N, H), jnp.float32),
            jax.ShapeDtypeStruct((N, H), jnp.float32),
        ],
    )(h, W1a, W1b)


def _sc_gather(Apk, Bpk, idx3r, idx3c):
    mesh = plsc.VectorSubcoreMesh(core_axis_name="c", subcore_axis_name="s")

    @functools.partial(
        pl.kernel,
        mesh=mesh,
        out_type=[
            jax.ShapeDtypeStruct((EP, H), jnp.float32),
            jax.ShapeDtypeStruct((EP, H), jnp.float32),
        ],
        scratch_types=[
            pltpu.VMEM((NCH_G, CH), jnp.int32),
            pltpu.VMEM((NCH_G, CH), jnp.int32),
            pltpu.VMEM((CH, H), jnp.float32),
            pltpu.VMEM((CH, H), jnp.float32),
            pltpu.SemaphoreType.DMA,
            pltpu.SemaphoreType.DMA,
        ],
    )
    def k(a_hbm, b_hbm, ir_hbm, ic_hbm, p_hbm, q_hbm, ir_v, ic_v, bufa, bufb,
          sema, semb):
        cid = lax.axis_index("c")
        sid = lax.axis_index("s")
        wid = sid * NC + cid
        base = wid * EPW
        pltpu.sync_copy(ir_hbm.at[wid], ir_v)
        pltpu.sync_copy(ic_hbm.at[wid], ic_v)

        def body(j, carry):
            off = base + j * CH
            cpa = pltpu.async_copy(a_hbm.at[ir_v.at[j]], bufa, sema)
            cpb = pltpu.async_copy(b_hbm.at[ic_v.at[j]], bufb, semb)
            cpa.wait()
            pltpu.sync_copy(bufa, p_hbm.at[pl.ds(off, CH)])
            cpb.wait()
            pltpu.sync_copy(bufb, q_hbm.at[pl.ds(off, CH)])
            return carry

        lax.fori_loop(0, NCH_G, body, 0)

    return k(Apk, Bpk, idx3r, idx3c)


def _edge_mlp(P, Q, ea2, cx2, cy2, cz2, w1c, b1r, W2, b2r, W3, I512):
    def body(p_ref, q_ref, ea_ref, cx_ref, cy_ref, cz_ref, w1c_ref,
             b1_ref, w2_ref, b2_ref, w3_ref, i_ref,
             tx_ref, ty_ref, tz_ref):
        ident = i_ref[...]
        ea = ea_ref[...]
        dn_outer = (((0,), (0,)), ((), ()))
        eaw = jnp.concatenate(
            [lax.dot_general(ea[r:r + 1, :], w1c_ref[...], dn_outer,
                             preferred_element_type=jnp.float32)
             for r in range(RB)], axis=0)

        s = p_ref[...] + q_ref[...] + eaw + b1_ref[...]
        x1 = jax.nn.silu(s)
        y = (jnp.dot(x1, w2_ref[...], preferred_element_type=jnp.float32)
             + b2_ref[...])
        x2 = jax.nn.silu(y)
        m_col = jnp.dot(x2, w3_ref[...], preferred_element_type=jnp.float32)

        dn_row = (((0,), (0,)), ((), ()))
        m_rows = jnp.concatenate(
            [lax.dot_general(m_col[r * 512:(r + 1) * 512, :], ident, dn_row,
                             preferred_element_type=jnp.float32)
             for r in range(RB)], axis=0)

        tx_ref[...] = cx_ref[...] * m_rows
        ty_ref[...] = cy_ref[...] * m_rows
        tz_ref[...] = cz_ref[...] * m_rows

    return pl.pallas_call(
        body,
        grid=(EP // BE,),
        in_specs=[
            pl.BlockSpec((BE, H), lambda i: (i, 0)),
            pl.BlockSpec((BE, H), lambda i: (i, 0)),
            pl.BlockSpec((RB, 512), lambda i: (i, 0)),
            pl.BlockSpec((RB, 512), lambda i: (i, 0)),
            pl.BlockSpec((RB, 512), lambda i: (i, 0)),
            pl.BlockSpec((RB, 512), lambda i: (i, 0)),
            pl.BlockSpec((1, H), lambda i: (0, 0)),
            pl.BlockSpec((1, H), lambda i: (0, 0)),
            pl.BlockSpec((H, H), lambda i: (0, 0)),
            pl.BlockSpec((1, H), lambda i: (0, 0)),
            pl.BlockSpec((H, 1), lambda i: (0, 0)),
            pl.BlockSpec((512, 512), lambda i: (0, 0)),
        ],
        out_specs=[
            pl.BlockSpec((RB, 512), lambda i: (i, 0)),
            pl.BlockSpec((RB, 512), lambda i: (i, 0)),
            pl.BlockSpec((RB, 512), lambda i: (i, 0)),
        ],
        out_shape=[
            jax.ShapeDtypeStruct((GROWS, 512), jnp.float32),
            jax.ShapeDtypeStruct((GROWS, 512), jnp.float32),
            jax.ShapeDtypeStruct((GROWS, 512), jnp.float32),
        ],
    )(P, Q, ea2, cx2, cy2, cz2, w1c, b1r, W2, b2r, W3, I512)


def _sc_scatter_finalize(tx2, ty2, tz2, idx3s, cx, cy, cz):
    mesh = plsc.VectorSubcoreMesh(core_axis_name="c", subcore_axis_name="s")

    @functools.partial(
        pl.kernel,
        mesh=mesh,
        out_type=[
            jax.ShapeDtypeStruct((N,), jnp.float32),
            jax.ShapeDtypeStruct((N,), jnp.float32),
            jax.ShapeDtypeStruct((N,), jnp.float32),
        ],
        scratch_types=[
            pltpu.VMEM((NCHS, 128), jnp.int32),
            pltpu.VMEM((RS, 512), jnp.float32),
            pltpu.VMEM((RS, 512), jnp.float32),
            pltpu.VMEM((RS, 512), jnp.float32),
            pltpu.VMEM((RPT,), jnp.float32),
            pltpu.VMEM((RPT,), jnp.float32),
            pltpu.VMEM((RPT,), jnp.float32),
            pltpu.VMEM_SHARED((N,), jnp.float32),
            pltpu.VMEM_SHARED((N,), jnp.float32),
            pltpu.VMEM_SHARED((N,), jnp.float32),
        ],
    )
    def k(tx_h, ty_h, tz_h, ix_h, cx_h, cy_h, cz_h, ox_h, oy_h, oz_h,
          ix_v, txv, tyv, tzv, avbuf, cbuf, obuf, accx, accy, accz):
        sid = lax.axis_index("s")
        rbase = sid * RS
        pltpu.sync_copy(ix_h.at[sid], ix_v)
        pltpu.sync_copy(tx_h.at[pl.ds(rbase, RS)], txv)
        pltpu.sync_copy(ty_h.at[pl.ds(rbase, RS)], tyv)
        pltpu.sync_copy(tz_h.at[pl.ds(rbase, RS)], tzv)

        def zb(i, carry):
            avbuf[pl.ds(i * 16, 16)] = jnp.zeros((16,), jnp.float32)
            return carry

        lax.fori_loop(0, RPT // 16, zb, 0)
        row0 = sid * RPT

        def zero_acc(nrows):
            pltpu.sync_copy(avbuf.at[pl.ds(0, nrows)], accx.at[pl.ds(row0, nrows)])
            pltpu.sync_copy(avbuf.at[pl.ds(0, nrows)], accy.at[pl.ds(row0, nrows)])
            pltpu.sync_copy(avbuf.at[pl.ds(0, nrows)], accz.at[pl.ds(row0, nrows)])

        @pl.when(sid < NS - 1)
        def _():
            zero_acc(RPT)

        @pl.when(sid == NS - 1)
        def _():
            zero_acc(N - (NS - 1) * RPT)

        plsc.subcore_barrier()

        def body(r, carry):
            for c in range(4):
                j = r * 4 + c
                sl = pl.ds(c * 128, 128)
                pltpu.sync_copy(txv.at[r, sl], accx.at[ix_v.at[j]], add=True)
                pltpu.sync_copy(tyv.at[r, sl], accy.at[ix_v.at[j]], add=True)
                pltpu.sync_copy(tzv.at[r, sl], accz.at[ix_v.at[j]], add=True)
            return carry

        lax.fori_loop(0, RS, body, 0)
        plsc.subcore_barrier()

        def fin(acc, c_h, o_h, nrows):
            pltpu.sync_copy(acc.at[pl.ds(row0, nrows)], avbuf.at[pl.ds(0, nrows)])
            pltpu.sync_copy(c_h.at[pl.ds(row0, nrows)], cbuf.at[pl.ds(0, nrows)])

            def fb(i, carry):
                sl = pl.ds(i * 16, 16)
                obuf[sl] = cbuf[sl] + avbuf[sl] * (1.0 / NORM)
                return carry

            lax.fori_loop(0, nrows // 16, fb, 0)
            pltpu.sync_copy(obuf.at[pl.ds(0, nrows)], o_h.at[pl.ds(row0, nrows)])

        def fin_all(nrows):
            fin(accx, cx_h, ox_h, nrows)
            fin(accy, cy_h, oy_h, nrows)
            fin(accz, cz_h, oz_h, nrows)

        @pl.when(sid < NS - 1)
        def _():
            fin_all(RPT)

        @pl.when(sid == NS - 1)
        def _():
            fin_all(N - (NS - 1) * RPT)

    return k(tx2, ty2, tz2, idx3s, cx, cy, cz)


def kernel(h, coord, edge_index, coord_diff, edge_attr, W1, b1, W2, b2, W3):
    f32 = jnp.float32
    row = edge_index[0].astype(jnp.int32)
    col = edge_index[1].astype(jnp.int32)
    pad = EP - E
    rowp = jnp.concatenate([row, jnp.zeros((pad,), jnp.int32)])
    colp = jnp.concatenate([col, jnp.zeros((pad,), jnp.int32)])

    W1a = W1[:H]
    W1b = W1[H:2 * H]
    w1c = W1[2 * H:2 * H + 1]

    A, B = _pre_node(h, W1a, W1b)

    idx3r = rowp.reshape(NW, NCH_G, CH)
    idx3c = colp.reshape(NW, NCH_G, CH)
    Pg, Qg = _sc_gather(A, B, idx3r, idx3c)

    zpad = jnp.zeros((pad,), f32)
    ea2 = jnp.concatenate([edge_attr[:, 0], zpad]).reshape(GROWS, 512)
    cx2 = jnp.concatenate([coord_diff[:, 0], zpad]).reshape(GROWS, 512)
    cy2 = jnp.concatenate([coord_diff[:, 1], zpad]).reshape(GROWS, 512)
    cz2 = jnp.concatenate([coord_diff[:, 2], zpad]).reshape(GROWS, 512)
    I512 = jnp.eye(512, dtype=f32)

    tx2, ty2, tz2 = _edge_mlp(
        Pg, Qg, ea2, cx2, cy2, cz2,
        w1c, b1.reshape(1, H), W2, b2.reshape(1, H), W3, I512)

    idx3s = rowp.reshape(NS, NCHS, 128)
    ox, oy, oz = _sc_scatter_finalize(tx2, ty2, tz2, idx3s,
                                      coord[:, 0], coord[:, 1], coord[:, 2])
    return jnp.stack([ox, oy, oz], axis=1)


# speedup vs baseline: 1.4846x; 1.4846x over previous
import functools

import jax
import jax.numpy as jnp
from jax import lax
from jax.experimental import pallas as pl
from jax.experimental.pallas import tpu as pltpu
from jax.experimental.pallas import tpu_sc as plsc

N = 10000
E = 320000
H = 128
NORM = 100.0

NC = 2
NS = 16
NW = NC * NS
EPW = E // NW
CH = 80
NCH_G = EPW // CH

BE = 2560
RB = BE // 512
GRID = E // BE
GROWS = E // 512

ES = E // NS
NCHS = ES // CH
RPT = 640


def _pre_node(h, W1a, W1b):
    def body(h_ref, wa_ref, wb_ref, a_ref, b_ref):
        hv = h_ref[...]
        a_ref[...] = jnp.dot(hv, wa_ref[...], preferred_element_type=jnp.float32)
        b_ref[...] = jnp.dot(hv, wb_ref[...], preferred_element_type=jnp.float32)

    BN = 2000
    return pl.pallas_call(
        body,
        grid=(N // BN,),
        in_specs=[
            pl.BlockSpec((BN, H), lambda i: (i, 0)),
            pl.BlockSpec((H, H), lambda i: (0, 0)),
            pl.BlockSpec((H, H), lambda i: (0, 0)),
        ],
        out_specs=[
            pl.BlockSpec((BN, H), lambda i: (i, 0)),
            pl.BlockSpec((BN, H), lambda i: (i, 0)),
        ],
        out_shape=[
            jax.ShapeDtypeStruct((N, H), jnp.float32),
            jax.ShapeDtypeStruct((N, H), jnp.float32),
        ],
    )(h, W1a, W1b)


def _sc_gather(A, B, idx3r, idx3c):
    mesh = plsc.VectorSubcoreMesh(core_axis_name="c", subcore_axis_name="s")

    @functools.partial(
        pl.kernel,
        mesh=mesh,
        out_type=[
            jax.ShapeDtypeStruct((E, H), jnp.float32),
            jax.ShapeDtypeStruct((E, H), jnp.float32),
        ],
        scratch_types=[
            pltpu.VMEM((NCH_G, CH), jnp.int32),
            pltpu.VMEM((NCH_G, CH), jnp.int32),
            pltpu.VMEM((CH, H), jnp.float32),
            pltpu.VMEM((CH, H), jnp.float32),
            pltpu.VMEM((CH, H), jnp.float32),
            pltpu.VMEM((CH, H), jnp.float32),
            pltpu.SemaphoreType.DMA,
            pltpu.SemaphoreType.DMA,
            pltpu.SemaphoreType.DMA,
            pltpu.SemaphoreType.DMA,
        ],
    )
    def k(a_hbm, b_hbm, ir_hbm, ic_hbm, p_hbm, q_hbm, ir_v, ic_v,
          bufa0, bufa1, bufb0, bufb1, sema0, sema1, semb0, semb1):
        cid = lax.axis_index("c")
        sid = lax.axis_index("s")
        wid = sid * NC + cid
        base = wid * EPW
        pltpu.sync_copy(ir_hbm.at[wid], ir_v)
        pltpu.sync_copy(ic_hbm.at[wid], ic_v)

        bufa = (bufa0, bufa1)
        bufb = (bufb0, bufb1)
        sema = (sema0, sema1)
        semb = (semb0, semb1)

        pltpu.async_copy(a_hbm.at[ir_v.at[0]], bufa0, sema0)
        pltpu.async_copy(b_hbm.at[ic_v.at[0]], bufb0, semb0)

        def body(jj, carry):
            for k2 in range(2):
                j = jj * 2 + k2
                cur = k2
                nxt = 1 - k2
                jn = j + 1

                @pl.when(jn < NCH_G)
                def _():
                    pltpu.async_copy(a_hbm.at[ir_v.at[jn]], bufa[nxt], sema[nxt])
                    pltpu.async_copy(b_hbm.at[ic_v.at[jn]], bufb[nxt], semb[nxt])

                off = base + j * CH
                pltpu.make_async_copy(a_hbm.at[ir_v.at[j]], bufa[cur],
                                      sema[cur]).wait()
                pltpu.sync_copy(bufa[cur], p_hbm.at[pl.ds(off, CH)])
                pltpu.make_async_copy(b_hbm.at[ic_v.at[j]], bufb[cur],
                                      semb[cur]).wait()
                pltpu.sync_copy(bufb[cur], q_hbm.at[pl.ds(off, CH)])
            return carry

        lax.fori_loop(0, NCH_G // 2, body, 0)
        jl = NCH_G - 1
        off = base + jl * CH
        pltpu.make_async_copy(a_hbm.at[ir_v.at[jl]], bufa[jl % 2],
                              sema[jl % 2]).wait()
        pltpu.sync_copy(bufa[jl % 2], p_hbm.at[pl.ds(off, CH)])
        pltpu.make_async_copy(b_hbm.at[ic_v.at[jl]], bufb[jl % 2],
                              semb[jl % 2]).wait()
        pltpu.sync_copy(bufb[jl % 2], q_hbm.at[pl.ds(off, CH)])

    return k(A, B, idx3r, idx3c)


def _edge_mlp(P, Q, cd, ea, w1c, b1r, W2, b2r, W3, I512):
    def body(p_ref, q_ref, cd_ref, ea_ref, w1c_ref,
             b1_ref, w2_ref, b2_ref, w3_ref, i_ref,
             tx_ref, ty_ref, tz_ref):
        ident = i_ref[...]
        eaw = jnp.dot(ea_ref[...], w1c_ref[...],
                      preferred_element_type=jnp.float32)

        s = p_ref[...] + q_ref[...] + eaw + b1_ref[...]
        x1 = jax.nn.silu(s)
        y = (jnp.dot(x1, w2_ref[...], preferred_element_type=jnp.float32)
             + b2_ref[...])
        x2 = jax.nn.silu(y)
        m_col = jnp.dot(x2, w3_ref[...], preferred_element_type=jnp.float32)

        dn_row = (((0,), (0,)), ((), ()))
        m_rows = jnp.concatenate(
            [lax.dot_general(m_col[r * 512:(r + 1) * 512, :], ident, dn_row,
                             preferred_element_type=jnp.float32)
             for r in range(RB)], axis=0)

        cd_full = cd_ref[...]
        cds = [lax.dot_general(cd_full[r * 512:(r + 1) * 512, :], ident,
                               dn_row, preferred_element_type=jnp.float32)
               for r in range(RB)]
        cx_rows = jnp.concatenate([c[0:1] for c in cds], axis=0)
        cy_rows = jnp.concatenate([c[1:2] for c in cds], axis=0)
        cz_rows = jnp.concatenate([c[2:3] for c in cds], axis=0)

        tx_ref[...] = (cx_rows * m_rows)[None]
        ty_ref[...] = (cy_rows * m_rows)[None]
        tz_ref[...] = (cz_rows * m_rows)[None]

    rspec = pl.BlockSpec((1, RB, 512), lambda i: (i, 0, 0))
    rshape = jax.ShapeDtypeStruct((GRID, RB, 512), jnp.float32)
    return pl.pallas_call(
        body,
        grid=(GRID,),
        in_specs=[
            pl.BlockSpec((BE, H), lambda i: (i, 0)),
            pl.BlockSpec((BE, H), lambda i: (i, 0)),
            pl.BlockSpec((BE, 3), lambda i: (i, 0)),
            pl.BlockSpec((BE, 1), lambda i: (i, 0)),
            pl.BlockSpec((1, H), lambda i: (0, 0)),
            pl.BlockSpec((1, H), lambda i: (0, 0)),
            pl.BlockSpec((H, H), lambda i: (0, 0)),
            pl.BlockSpec((1, H), lambda i: (0, 0)),
            pl.BlockSpec((H, 1), lambda i: (0, 0)),
            pl.BlockSpec((512, 512), lambda i: (0, 0)),
        ],
        out_specs=[rspec, rspec, rspec],
        out_shape=[rshape, rshape, rshape],
    )(P, Q, cd, ea, w1c, b1r, W2, b2r, W3, I512)


def _sc_scatter_finalize(tx1, ty1, tz1, idx3s, cx, cy, cz):
    mesh = plsc.VectorSubcoreMesh(core_axis_name="c", subcore_axis_name="s")

    @functools.partial(
        pl.kernel,
        mesh=mesh,
        out_type=[
            jax.ShapeDtypeStruct((N,), jnp.float32),
            jax.ShapeDtypeStruct((N,), jnp.float32),
            jax.ShapeDtypeStruct((N,), jnp.float32),
        ],
        scratch_types=[
            pltpu.VMEM((NCHS, CH), jnp.int32),
            pltpu.VMEM((ES,), jnp.float32),
            pltpu.VMEM((ES,), jnp.float32),
            pltpu.VMEM((ES,), jnp.float32),
            pltpu.VMEM((RPT,), jnp.float32),
            pltpu.VMEM((RPT,), jnp.float32),
            pltpu.VMEM((RPT,), jnp.float32),
            pltpu.VMEM_SHARED((N,), jnp.float32),
            pltpu.VMEM_SHARED((N,), jnp.float32),
            pltpu.VMEM_SHARED((N,), jnp.float32),
        ],
    )
    def k(tx_h, ty_h, tz_h, ix_h, cx_h, cy_h, cz_h, ox_h, oy_h, oz_h,
          ix_v, txv, tyv, tzv, avbuf, cbuf, obuf, accx, accy, accz):
        sid = lax.axis_index("s")
        base = sid * ES
        pltpu.sync_copy(ix_h.at[sid], ix_v)
        pltpu.sync_copy(tx_h.at[pl.ds(base, ES)], txv)
        pltpu.sync_copy(ty_h.at[pl.ds(base, ES)], tyv)
        pltpu.sync_copy(tz_h.at[pl.ds(base, ES)], tzv)

        def zb(i, carry):
            avbuf[pl.ds(i * 16, 16)] = jnp.zeros((16,), jnp.float32)
            return carry

        lax.fori_loop(0, RPT // 16, zb, 0)
        row0 = sid * RPT

        def zero_acc(nrows):
            pltpu.sync_copy(avbuf.at[pl.ds(0, nrows)], accx.at[pl.ds(row0, nrows)])
            pltpu.sync_copy(avbuf.at[pl.ds(0, nrows)], accy.at[pl.ds(row0, nrows)])
            pltpu.sync_copy(avbuf.at[pl.ds(0, nrows)], accz.at[pl.ds(row0, nrows)])

        @pl.when(sid < NS - 1)
        def _():
            zero_acc(RPT)

        @pl.when(sid == NS - 1)
        def _():
            zero_acc(N - (NS - 1) * RPT)

        plsc.subcore_barrier()

        def body(j, carry):
            src = pl.ds(j * CH, CH)
            ixr = ix_v.at[j]
            pltpu.sync_copy(txv.at[src], accx.at[ixr], add=True)
            pltpu.sync_copy(tyv.at[src], accy.at[ixr], add=True)
            pltpu.sync_copy(tzv.at[src], accz.at[ixr], add=True)
            return carry

        lax.fori_loop(0, NCHS, body, 0)
        plsc.subcore_barrier()

        def fin(acc, c_h, o_h, nrows):
            pltpu.sync_copy(acc.at[pl.ds(row0, nrows)], avbuf.at[pl.ds(0, nrows)])
            pltpu.sync_copy(c_h.at[pl.ds(row0, nrows)], cbuf.at[pl.ds(0, nrows)])

            def fb(i, carry):
                sl = pl.ds(i * 16, 16)
                obuf[sl] = cbuf[sl] + avbuf[sl] * (1.0 / NORM)
                return carry

            lax.fori_loop(0, nrows // 16, fb, 0)
            pltpu.sync_copy(obuf.at[pl.ds(0, nrows)], o_h.at[pl.ds(row0, nrows)])

        def fin_all(nrows):
            fin(accx, cx_h, ox_h, nrows)
            fin(accy, cy_h, oy_h, nrows)
            fin(accz, cz_h, oz_h, nrows)

        @pl.when(sid < NS - 1)
        def _():
            fin_all(RPT)

        @pl.when(sid == NS - 1)
        def _():
            fin_all(N - (NS - 1) * RPT)

    return k(tx1, ty1, tz1, idx3s, cx, cy, cz)


def kernel(h, coord, edge_index, coord_diff, edge_attr, W1, b1, W2, b2, W3):
    f32 = jnp.float32
    row = edge_index[0].astype(jnp.int32)
    col = edge_index[1].astype(jnp.int32)

    W1a = W1[:H]
    W1b = W1[H:2 * H]
    w1c = W1[2 * H:2 * H + 1]
    I512 = jnp.eye(512, dtype=f32)

    A, B = _pre_node(h, W1a, W1b)

    idx3r = row.reshape(NW, NCH_G, CH)
    idx3c = col.reshape(NW, NCH_G, CH)
    Pg, Qg = _sc_gather(A, B, idx3r, idx3c)

    tx3, ty3, tz3 = _edge_mlp(
        Pg, Qg, coord_diff, edge_attr,
        w1c, b1.reshape(1, H), W2, b2.reshape(1, H), W3, I512)

    tx1 = tx3.reshape(E)
    ty1 = ty3.reshape(E)
    tz1 = tz3.reshape(E)
    idx3s = row.reshape(NS, NCHS, CH)
    ox, oy, oz = _sc_scatter_finalize(tx1, ty1, tz1, idx3s,
                                      coord[:, 0], coord[:, 1], coord[:, 2])
    return jnp.stack([ox, oy, oz], axis=1)


# speedup vs baseline: 1.8059x; 1.2164x over previous
import functools

import jax
import jax.numpy as jnp
from jax import lax
from jax.experimental import pallas as pl
from jax.experimental.pallas import tpu as pltpu
from jax.experimental.pallas import tpu_sc as plsc

N = 10000
E = 320000
H = 128
NORM = 100.0

NC = 2
NS = 16
CH = 80

BE = 2560
RB = BE // 512
GRID = E // BE
GROWS = E // 512

ES = E // NS
NCHS = ES // CH
RPT = 640


def _pre_node(h, W1a, W1b):
    def body(h_ref, wa_ref, wb_ref, a_ref, b_ref):
        hv = h_ref[...]
        a_ref[...] = jnp.dot(hv, wa_ref[...], preferred_element_type=jnp.float32)
        b_ref[...] = jnp.dot(hv, wb_ref[...], preferred_element_type=jnp.float32)

    BN = 2000
    return pl.pallas_call(
        body,
        grid=(N // BN,),
        in_specs=[
            pl.BlockSpec((BN, H), lambda i: (i, 0)),
            pl.BlockSpec((H, H), lambda i: (0, 0)),
            pl.BlockSpec((H, H), lambda i: (0, 0)),
        ],
        out_specs=[
            pl.BlockSpec((BN, H), lambda i: (i, 0)),
            pl.BlockSpec((BN, H), lambda i: (i, 0)),
        ],
        out_shape=[
            jax.ShapeDtypeStruct((N, H), jnp.float32),
            jax.ShapeDtypeStruct((N, H), jnp.float32),
        ],
    )(h, W1a, W1b)


def _sc_gather(A, B, idx4r, idx4c):
    mesh = plsc.VectorSubcoreMesh(core_axis_name="c", subcore_axis_name="s")
    NHF = NCHS // 2

    @functools.partial(
        pl.kernel,
        mesh=mesh,
        out_type=[
            jax.ShapeDtypeStruct((E, H), jnp.float32),
            jax.ShapeDtypeStruct((E, H), jnp.float32),
        ],
        scratch_types=[
            pltpu.VMEM((NHF, CH), jnp.int32),
            pltpu.VMEM((CH, H), jnp.float32),
            pltpu.VMEM((CH, H), jnp.float32),
            pltpu.VMEM_SHARED((N, H), jnp.float32),
            pltpu.SemaphoreType.DMA,
            pltpu.SemaphoreType.DMA,
        ],
    )
    def k(a_hbm, b_hbm, ir_hbm, ic_hbm, p_hbm, q_hbm, ix_v, buf0, buf1,
          table, sem0, sem1):
        cid = lax.axis_index("c")
        sid = lax.axis_index("s")

        def run(t_hbm, i_hbm, o_hbm):
            row0 = sid * RPT

            @pl.when(sid < NS - 1)
            def _():
                pltpu.sync_copy(t_hbm.at[pl.ds(row0, RPT)],
                                table.at[pl.ds(row0, RPT)])

            @pl.when(sid == NS - 1)
            def _():
                last = N - (NS - 1) * RPT
                pltpu.sync_copy(t_hbm.at[pl.ds(row0, last)],
                                table.at[pl.ds(row0, last)])

            plsc.subcore_barrier()

            base = sid * ES
            bufs = (buf0, buf1)
            sems = (sem0, sem1)

            for hf in range(2):
                pltpu.sync_copy(i_hbm.at[sid, hf], ix_v)
                cb = hf * NHF
                pltpu.async_copy(table.at[ix_v.at[0]], buf0, sem0)

                def body(jj, carry):
                    for k2 in range(2):
                        j = jj * 2 + k2
                        cur = k2
                        nxt = 1 - k2
                        jn = j + 1

                        @pl.when(jn < NHF)
                        def _():
                            pltpu.async_copy(table.at[ix_v.at[jn]],
                                             bufs[nxt], sems[nxt])

                        off = base + (cb + j) * CH
                        pltpu.make_async_copy(table.at[ix_v.at[j]],
                                              bufs[cur], sems[cur]).wait()
                        pltpu.sync_copy(bufs[cur], o_hbm.at[pl.ds(off, CH)])
                    return carry

                lax.fori_loop(0, NHF // 2, body, 0)
                jl = NHF - 1
                off = base + (cb + jl) * CH
                pltpu.make_async_copy(table.at[ix_v.at[jl]], bufs[jl % 2],
                                      sems[jl % 2]).wait()
                pltpu.sync_copy(bufs[jl % 2], o_hbm.at[pl.ds(off, CH)])

        @pl.when(cid == 0)
        def _():
            run(a_hbm, ir_hbm, p_hbm)

        @pl.when(cid == 1)
        def _():
            run(b_hbm, ic_hbm, q_hbm)

    return k(A, B, idx4r, idx4c)


def _edge_mlp(P, Q, cd, ea, w1c, b1r, W2, b2r, W3, I512):
    def body(p_ref, q_ref, cd_ref, ea_ref, w1c_ref,
             b1_ref, w2_ref, b2_ref, w3_ref, i_ref,
             tx_ref, ty_ref, tz_ref):
        ident = i_ref[...]
        eaw = jnp.dot(ea_ref[...], w1c_ref[...],
                      preferred_element_type=jnp.float32)

        s = p_ref[...] + q_ref[...] + eaw + b1_ref[...]
        x1 = jax.nn.silu(s)
        y = (jnp.dot(x1, w2_ref[...], preferred_element_type=jnp.float32)
             + b2_ref[...])
        x2 = jax.nn.silu(y)
        m_col = jnp.dot(x2, w3_ref[...], preferred_element_type=jnp.float32)

        dn_row = (((0,), (0,)), ((), ()))
        m_rows = jnp.concatenate(
            [lax.dot_general(m_col[r * 512:(r + 1) * 512, :], ident, dn_row,
                             preferred_element_type=jnp.float32)
             for r in range(RB)], axis=0)

        cd_full = cd_ref[...]
        cds = [lax.dot_general(cd_full[r * 512:(r + 1) * 512, :], ident,
                               dn_row, preferred_element_type=jnp.float32)
               for r in range(RB)]
        cx_rows = jnp.concatenate([c[0:1] for c in cds], axis=0)
        cy_rows = jnp.concatenate([c[1:2] for c in cds], axis=0)
        cz_rows = jnp.concatenate([c[2:3] for c in cds], axis=0)

        tx_ref[...] = (cx_rows * m_rows)[None]
        ty_ref[...] = (cy_rows * m_rows)[None]
        tz_ref[...] = (cz_rows * m_rows)[None]

    rspec = pl.BlockSpec((1, RB, 512), lambda i: (i, 0, 0))
    rshape = jax.ShapeDtypeStruct((GRID, RB, 512), jnp.float32)
    return pl.pallas_call(
        body,
        grid=(GRID,),
        in_specs=[
            pl.BlockSpec((BE, H), lambda i: (i, 0)),
            pl.BlockSpec((BE, H), lambda i: (i, 0)),
            pl.BlockSpec((BE, 3), lambda i: (i, 0)),
            pl.BlockSpec((BE, 1), lambda i: (i, 0)),
            pl.BlockSpec((1, H), lambda i: (0, 0)),
            pl.BlockSpec((1, H), lambda i: (0, 0)),
            pl.BlockSpec((H, H), lambda i: (0, 0)),
            pl.BlockSpec((1, H), lambda i: (0, 0)),
            pl.BlockSpec((H, 1), lambda i: (0, 0)),
            pl.BlockSpec((512, 512), lambda i: (0, 0)),
        ],
        out_specs=[rspec, rspec, rspec],
        out_shape=[rshape, rshape, rshape],
    )(P, Q, cd, ea, w1c, b1r, W2, b2r, W3, I512)


def _sc_scatter_finalize(tx1, ty1, tz1, idx3s, cx, cy, cz):
    mesh = plsc.VectorSubcoreMesh(core_axis_name="c", subcore_axis_name="s")

    @functools.partial(
        pl.kernel,
        mesh=mesh,
        out_type=[
            jax.ShapeDtypeStruct((N,), jnp.float32),
            jax.ShapeDtypeStruct((N,), jnp.float32),
            jax.ShapeDtypeStruct((N,), jnp.float32),
        ],
        scratch_types=[
            pltpu.VMEM((NCHS, CH), jnp.int32),
            pltpu.VMEM((ES,), jnp.float32),
            pltpu.VMEM((ES,), jnp.float32),
            pltpu.VMEM((ES,), jnp.float32),
            pltpu.VMEM((RPT,), jnp.float32),
            pltpu.VMEM((RPT,), jnp.float32),
            pltpu.VMEM((RPT,), jnp.float32),
            pltpu.VMEM_SHARED((N,), jnp.float32),
            pltpu.VMEM_SHARED((N,), jnp.float32),
            pltpu.VMEM_SHARED((N,), jnp.float32),
        ],
    )
    def k(tx_h, ty_h, tz_h, ix_h, cx_h, cy_h, cz_h, ox_h, oy_h, oz_h,
          ix_v, txv, tyv, tzv, avbuf, cbuf, obuf, accx, accy, accz):
        sid = lax.axis_index("s")
        base = sid * ES
        pltpu.sync_copy(ix_h.at[sid], ix_v)
        pltpu.sync_copy(tx_h.at[pl.ds(base, ES)], txv)
        pltpu.sync_copy(ty_h.at[pl.ds(base, ES)], tyv)
        pltpu.sync_copy(tz_h.at[pl.ds(base, ES)], tzv)

        def zb(i, carry):
            avbuf[pl.ds(i * 16, 16)] = jnp.zeros((16,), jnp.float32)
            return carry

        lax.fori_loop(0, RPT // 16, zb, 0)
        row0 = sid * RPT

        def zero_acc(nrows):
            pltpu.sync_copy(avbuf.at[pl.ds(0, nrows)], accx.at[pl.ds(row0, nrows)])
            pltpu.sync_copy(avbuf.at[pl.ds(0, nrows)], accy.at[pl.ds(row0, nrows)])
            pltpu.sync_copy(avbuf.at[pl.ds(0, nrows)], accz.at[pl.ds(row0, nrows)])

        @pl.when(sid < NS - 1)
        def _():
            zero_acc(RPT)

        @pl.when(sid == NS - 1)
        def _():
            zero_acc(N - (NS - 1) * RPT)

        plsc.subcore_barrier()

        def body(j, carry):
            src = pl.ds(j * CH, CH)
            ixr = ix_v.at[j]
            pltpu.sync_copy(txv.at[src], accx.at[ixr], add=True)
            pltpu.sync_copy(tyv.at[src], accy.at[ixr], add=True)
            pltpu.sync_copy(tzv.at[src], accz.at[ixr], add=True)
            return carry

        lax.fori_loop(0, NCHS, body, 0)
        plsc.subcore_barrier()

        def fin(acc, c_h, o_h, nrows):
            pltpu.sync_copy(acc.at[pl.ds(row0, nrows)], avbuf.at[pl.ds(0, nrows)])
            pltpu.sync_copy(c_h.at[pl.ds(row0, nrows)], cbuf.at[pl.ds(0, nrows)])

            def fb(i, carry):
                sl = pl.ds(i * 16, 16)
                obuf[sl] = cbuf[sl] + avbuf[sl] * (1.0 / NORM)
                return carry

            lax.fori_loop(0, nrows // 16, fb, 0)
            pltpu.sync_copy(obuf.at[pl.ds(0, nrows)], o_h.at[pl.ds(row0, nrows)])

        def fin_all(nrows):
            fin(accx, cx_h, ox_h, nrows)
            fin(accy, cy_h, oy_h, nrows)
            fin(accz, cz_h, oz_h, nrows)

        @pl.when(sid < NS - 1)
        def _():
            fin_all(RPT)

        @pl.when(sid == NS - 1)
        def _():
            fin_all(N - (NS - 1) * RPT)

    return k(tx1, ty1, tz1, idx3s, cx, cy, cz)


def kernel(h, coord, edge_index, coord_diff, edge_attr, W1, b1, W2, b2, W3):
    f32 = jnp.float32
    row = edge_index[0].astype(jnp.int32)
    col = edge_index[1].astype(jnp.int32)

    W1a = W1[:H]
    W1b = W1[H:2 * H]
    w1c = W1[2 * H:2 * H + 1]
    I512 = jnp.eye(512, dtype=f32)

    A, B = _pre_node(h, W1a, W1b)

    idx4r = row.reshape(NS, 2, NCHS // 2, CH)
    idx4c = col.reshape(NS, 2, NCHS // 2, CH)
    Pg, Qg = _sc_gather(A, B, idx4r, idx4c)

    tx3, ty3, tz3 = _edge_mlp(
        Pg, Qg, coord_diff, edge_attr,
        w1c, b1.reshape(1, H), W2, b2.reshape(1, H), W3, I512)

    tx1 = tx3.reshape(E)
    ty1 = ty3.reshape(E)
    tz1 = tz3.reshape(E)
    idx3s = row.reshape(NS, NCHS, CH)
    ox, oy, oz = _sc_scatter_finalize(tx1, ty1, tz1, idx3s,
                                      coord[:, 0], coord[:, 1], coord[:, 2])
    return jnp.stack([ox, oy, oz], axis=1)
